# Initial kernel scaffold; baseline (speedup 1.0000x reference)
import jax
import jax.numpy as jnp
from jax.experimental import pallas as pl


def kernel(prob):
    raise NotImplementedError("write your pallas kernel here")


# speedup vs baseline: 23.5040x; 23.5040x over previous
import functools

import jax
import jax.numpy as jnp
from jax.experimental import pallas as pl
from jax.experimental.pallas import tpu as pltpu

_NC = 19
_RATIO = 0.2


def _loss_kernel(x_ref, loss_ref, cnt_ref, val_ref, acc_ref, *, nt, scale):
    n = pl.program_id(0)
    t = pl.program_id(1)

    x0 = x_ref[0, 0]
    m = x0
    s = x0 * x0
    idx = jnp.zeros(x0.shape, jnp.int32)
    for c in range(1, _NC):
        v = x_ref[0, c]
        s = s + v * v
        upd = v > m
        m = jnp.where(upd, v, m)
        idx = jnp.where(upd, c, idx)

    cnts = []
    vals = []
    for c in range(_NC):
        eq = idx == c
        cnts.append(jnp.sum(eq.astype(jnp.float32)))
        vals.append(jnp.sum(jnp.where(eq, s, 0.0)))
    cnt_vec = jnp.stack(cnts)
    val_vec = jnp.stack(vals)

    @pl.when(t == 0)
    def _():
        cnt_ref[0, :] = cnt_vec
        val_ref[0, :] = val_vec

    @pl.when(t != 0)
    def _():
        cnt_ref[0, :] = cnt_ref[0, :] + cnt_vec
        val_ref[0, :] = val_ref[0, :] + val_vec

    @pl.when(t == nt - 1)
    def _():
        hist = cnt_ref[0, :]
        acc_val = val_ref[0, :]
        tot = jnp.sum(hist)
        powh = jnp.where(
            hist > 0.0,
            jnp.exp(_RATIO * jnp.log(jnp.maximum(hist, 1.0))),
            0.0,
        )
        powt = jnp.exp((1.0 - _RATIO) * jnp.log(tot))
        denom = jnp.maximum(powh * powt, 1.0)
        contrib = jnp.sum(acc_val / denom)
        prev = jnp.where(n == 0, 0.0, acc_ref[0])
        acc = prev + contrib
        acc_ref[0] = acc

        @pl.when(n == pl.num_programs(0) - 1)
        def _():
            loss_ref[:, :] = jnp.full((1, 1), acc * scale, jnp.float32)


def kernel(prob):
    N, C, H, W = prob.shape
    TH = 128
    nt = H // TH
    out = pl.pallas_call(
        functools.partial(_loss_kernel, nt=nt, scale=-1.0 / (N * C)),
        grid=(N, nt),
        in_specs=[pl.BlockSpec((1, C, TH, W), lambda n, t: (n, 0, t, 0))],
        out_specs=pl.BlockSpec((1, 1), lambda n, t: (0, 0)),
        out_shape=jax.ShapeDtypeStruct((1, 1), jnp.float32),
        scratch_shapes=[
            pltpu.VMEM((1, _NC), jnp.float32),
            pltpu.VMEM((1, _NC), jnp.float32),
            pltpu.SMEM((1,), jnp.float32),
        ],
    )(prob)
    return out[0, 0]
